# Initial kernel scaffold; baseline (speedup 1.0000x reference)
import jax
import jax.numpy as jnp
from jax.experimental import pallas as pl


def kernel(x, Wqkv, Wo, Wiq, Wik, Wiw, Wg1, bg1, Wg2, bg2):
    raise NotImplementedError("write your pallas kernel here")


# speedup vs baseline: 36.0018x; 36.0018x over previous
import functools

import numpy as np

import jax
import jax.numpy as jnp
from jax.experimental import pallas as pl
from jax.experimental.pallas import tpu as pltpu

D_MODEL = 1024
N_HEADS = 16
D_K = 64
HALF = 32
IND_HEADS = 4
IND_DIM = 64
SEQ = 2048
TILE = 128
N_TILES = SEQ // TILE
TOPK_BASE = 512.0
GMIN = 0.5
GMAX = 1.5
ROPE_B = 10000.0

_NEG = -1e30


def _i32(v):
    return int(np.uint32(v & 0xFFFFFFFF).view(np.int32))


def _proj_kernel(x_ref, w_ref, cos_ref, sin_ref,
                 q_ref, kt_ref, v_ref, iq_ref, ikt_ref, iw_ref):
    xw = jnp.dot(x_ref[...], w_ref[...], preferred_element_type=jnp.float32)
    c = cos_ref[...]
    s = sin_ref[...]
    q1 = xw[:, 0:512]
    q2 = xw[:, 512:1024]
    k1 = xw[:, 1024:1536]
    k2 = xw[:, 1536:2048]
    q_ref[:, 0:512] = q1 * c - q2 * s
    q_ref[:, 512:1024] = q2 * c + q1 * s
    kt_ref[0:512, :] = (k1 * c - k2 * s).T
    kt_ref[512:1024, :] = (k2 * c + k1 * s).T
    v_ref[...] = xw[:, 2048:3072]
    iq_ref[...] = xw[:, 3072:3328]
    ikt_ref[...] = xw[:, 3328:3392].T
    iw_ref[...] = xw[:, 3392:3456]


def _run_proj(x2, wcat, cos_t, sin_t):
    return pl.pallas_call(
        _proj_kernel,
        grid=(N_TILES,),
        in_specs=[
            pl.BlockSpec((TILE, D_MODEL), lambda i: (i, 0)),
            pl.BlockSpec((D_MODEL, 3456), lambda i: (0, 0)),
            pl.BlockSpec((TILE, 512), lambda i: (i, 0)),
            pl.BlockSpec((TILE, 512), lambda i: (i, 0)),
        ],
        out_specs=[
            pl.BlockSpec((TILE, D_MODEL), lambda i: (i, 0)),
            pl.BlockSpec((D_MODEL, TILE), lambda i: (0, i)),
            pl.BlockSpec((TILE, D_MODEL), lambda i: (i, 0)),
            pl.BlockSpec((TILE, 256), lambda i: (i, 0)),
            pl.BlockSpec((IND_DIM, TILE), lambda i: (0, i)),
            pl.BlockSpec((TILE, 64), lambda i: (i, 0)),
        ],
        out_shape=[
            jax.ShapeDtypeStruct((SEQ, D_MODEL), jnp.float32),
            jax.ShapeDtypeStruct((D_MODEL, SEQ), jnp.float32),
            jax.ShapeDtypeStruct((SEQ, D_MODEL), jnp.float32),
            jax.ShapeDtypeStruct((SEQ, 256), jnp.float32),
            jax.ShapeDtypeStruct((IND_DIM, SEQ), jnp.float32),
            jax.ShapeDtypeStruct((SEQ, 64), jnp.float32),
        ],
    )(x2, wcat, cos_t, sin_t)


def _scores_kernel(iq_ref, ikt_ref, iw_ref, sc_ref, ent_ref):
    ti = pl.program_id(0)
    ikt = ikt_ref[...]
    acc = jnp.zeros((TILE, SEQ), jnp.float32)
    for h in range(IND_HEADS):
        iqh = iq_ref[:, h * IND_DIM:(h + 1) * IND_DIM]
        dots = jnp.dot(iqh, ikt, preferred_element_type=jnp.float32)
        acc = acc + jnp.maximum(dots, 0.0) * iw_ref[:, h:h + 1]
    rows = ti * TILE + jax.lax.broadcasted_iota(jnp.int32, (TILE, SEQ), 0)
    cols = jax.lax.broadcasted_iota(jnp.int32, (TILE, SEQ), 1)
    masked = jnp.where(cols > rows, -jnp.inf, acc)
    sc_ref[...] = masked
    m = jnp.max(masked, axis=1, keepdims=True)
    e = jnp.exp(masked - m)
    z = jnp.sum(e, axis=1, keepdims=True)
    p = e / z
    ent_rows = -jnp.sum(p * jnp.log(p + 1e-9), axis=1)

    @pl.when(ti == 0)
    def _():
        ent_ref[0, 0] = 0.0

    ent_ref[0, 0] += jnp.sum(ent_rows)


def _run_scores(iq, ikt, iw):
    return pl.pallas_call(
        _scores_kernel,
        grid=(N_TILES,),
        in_specs=[
            pl.BlockSpec((TILE, 256), lambda i: (i, 0)),
            pl.BlockSpec((IND_DIM, SEQ), lambda i: (0, 0)),
            pl.BlockSpec((TILE, 64), lambda i: (i, 0)),
        ],
        out_specs=[
            pl.BlockSpec((TILE, SEQ), lambda i: (i, 0)),
            pl.BlockSpec(memory_space=pltpu.SMEM),
        ],
        out_shape=[
            jax.ShapeDtypeStruct((SEQ, SEQ), jnp.float32),
            jax.ShapeDtypeStruct((1, 1), jnp.float32),
        ],
    )(iq, ikt, iw)


def _gate_kernel(x_ref, w1_ref, b1_ref, w2_ref, b2_ref, g_ref):
    pooled = jnp.mean(x_ref[...], axis=0, keepdims=True)
    h1 = jnp.maximum(
        jnp.dot(pooled, w1_ref[...], preferred_element_type=jnp.float32)
        + b1_ref[...], 0.0)
    o = jnp.dot(h1, w2_ref[...], preferred_element_type=jnp.float32) + b2_ref[...]
    g_ref[0, 0] = jax.nn.sigmoid(o)[0, 0]


def _run_gate(x2, wg1, bg1, wg2, bg2):
    gh = wg1.shape[1]
    return pl.pallas_call(
        _gate_kernel,
        in_specs=[
            pl.BlockSpec((SEQ, D_MODEL), lambda: (0, 0)),
            pl.BlockSpec((D_MODEL, gh), lambda: (0, 0)),
            pl.BlockSpec((1, gh), lambda: (0, 0)),
            pl.BlockSpec((gh, 1), lambda: (0, 0)),
            pl.BlockSpec((1, 1), lambda: (0, 0)),
        ],
        out_specs=pl.BlockSpec(memory_space=pltpu.SMEM),
        out_shape=jax.ShapeDtypeStruct((1, 1), jnp.float32),
    )(x2, wg1, bg1.reshape(1, gh), wg2, bg2.reshape(1, 1))


def _select_kernel(k_ref, sc_ref, mask_ref):
    kval = k_ref[0, 0]
    sv = sc_ref[...] + 0.0
    b = jax.lax.bitcast_convert_type(sv, jnp.int32)
    sign_bit = jnp.int32(_i32(0x80000000))
    u = jnp.where(b < 0, jnp.bitwise_not(b), jnp.bitwise_or(b, sign_bit))
    p_hi = jnp.zeros((TILE, 1), jnp.int32)
    k_rem = jnp.full((TILE, 1), kval, jnp.int32)
    for j in range(31, -1, -1):
        bit = jnp.int32(_i32(1 << j))
        himask = jnp.int32(_i32(~((1 << (j + 1)) - 1)))
        cand = (u & himask) == p_hi
        is1 = (u & bit) != 0
        cnt1 = jnp.sum((cand & is1).astype(jnp.int32), axis=1, keepdims=True)
        take = cnt1 >= k_rem
        p_hi = jnp.where(take, p_hi | bit, p_hi)
        k_rem = jnp.where(take, k_rem, k_rem - cnt1)
    key_s = u ^ sign_bit
    t_s = p_hi ^ sign_bit
    eq = u == p_hi
    g_cnt = jnp.sum((key_s > t_s).astype(jnp.int32), axis=1, keepdims=True)
    quota = kval - g_cnt
    idx = jax.lax.broadcasted_iota(jnp.int32, (TILE, SEQ), 1)
    q_hi = jnp.zeros((TILE, 1), jnp.int32)
    for j in range(10, -1, -1):
        bit = jnp.int32(1 << j)
        himask = jnp.int32(_i32(~((1 << (j + 1)) - 1)))
        cand = eq & ((idx & himask) == q_hi)
        c0 = jnp.sum((cand & ((idx & bit) == 0)).astype(jnp.int32),
                     axis=1, keepdims=True)
        take0 = quota <= c0
        q_hi = jnp.where(take0, q_hi, q_hi | bit)
        quota = jnp.where(take0, quota, quota - c0)
    allowed = (key_s > t_s) | (eq & (idx <= q_hi))
    mask_ref[...] = allowed.astype(jnp.int8)


def _run_select(scores, kscal):
    return pl.pallas_call(
        _select_kernel,
        grid=(N_TILES,),
        in_specs=[
            pl.BlockSpec(memory_space=pltpu.SMEM),
            pl.BlockSpec((TILE, SEQ), lambda i: (i, 0)),
        ],
        out_specs=pl.BlockSpec((TILE, SEQ), lambda i: (i, 0)),
        out_shape=jax.ShapeDtypeStruct((SEQ, SEQ), jnp.int8),
    )(kscal, scores)


def _attn_kernel(q_ref, kt_ref, v_ref, mask_ref, wo_ref, o_ref, acc_ref):
    ok = mask_ref[...] != 0
    scale = jnp.float32(1.0 / np.sqrt(D_K))
    for h in range(N_HEADS):
        q1 = q_ref[:, h * HALF:(h + 1) * HALF]
        q2 = q_ref[:, 512 + h * HALF:512 + (h + 1) * HALF]
        kt1 = kt_ref[h * HALF:(h + 1) * HALF, :]
        kt2 = kt_ref[512 + h * HALF:512 + (h + 1) * HALF, :]
        logits = (jnp.dot(q1, kt1, preferred_element_type=jnp.float32)
                  + jnp.dot(q2, kt2, preferred_element_type=jnp.float32)) * scale
        logits = jnp.where(ok, logits, _NEG)
        m = jnp.max(logits, axis=1, keepdims=True)
        e = jnp.exp(logits - m)
        z = jnp.sum(e, axis=1, keepdims=True)
        p = e / z
        vh = v_ref[:, h * D_K:(h + 1) * D_K]
        acc_ref[:, h * D_K:(h + 1) * D_K] = jnp.dot(
            p, vh, preferred_element_type=jnp.float32)
    o_ref[...] = jnp.dot(acc_ref[...], wo_ref[...],
                         preferred_element_type=jnp.float32)


def _run_attn(q, kt, v, mask, wo):
    return pl.pallas_call(
        _attn_kernel,
        grid=(N_TILES,),
        in_specs=[
            pl.BlockSpec((TILE, D_MODEL), lambda i: (i, 0)),
            pl.BlockSpec((D_MODEL, SEQ), lambda i: (0, 0)),
            pl.BlockSpec((SEQ, D_MODEL), lambda i: (0, 0)),
            pl.BlockSpec((TILE, SEQ), lambda i: (i, 0)),
            pl.BlockSpec((D_MODEL, D_MODEL), lambda i: (0, 0)),
        ],
        out_specs=pl.BlockSpec((TILE, D_MODEL), lambda i: (i, 0)),
        out_shape=jax.ShapeDtypeStruct((SEQ, D_MODEL), jnp.float32),
        scratch_shapes=[pltpu.VMEM((TILE, D_MODEL), jnp.float32)],
    )(q, kt, v, mask, wo)


def _two_prod_(a, b):
    p = a * b
    c = jnp.float32(4097.0)
    a_c = a * c
    a_hi = a_c - (a_c - a)
    a_lo = a - a_hi
    b_c = b * c
    b_hi = b_c - (b_c - b)
    b_lo = b - b_hi
    err = ((a_hi * b_hi - p) + a_hi * b_lo + a_lo * b_hi) + a_lo * b_lo
    return p, err


def _exact_floor_prod_(a, b):
    p, err = _two_prod_(a, b)
    base = jnp.floor(p)
    r = p - base
    t = r + err
    base = base + jnp.where(t >= 1.0, 1.0, 0.0) - jnp.where(t < 0.0, 1.0, 0.0)
    return base


def _perm_halfsplit():
    j = np.arange(512)
    h = j // HALF
    i = j % HALF
    return np.concatenate([h * D_K + 2 * i, h * D_K + 2 * i + 1])


def kernel(x, Wqkv, Wo, Wiq, Wik, Wiw, Wg1, bg1, Wg2, bg2):
    b, s, d = x.shape
    x2 = x[0]

    perm = _perm_halfsplit()
    wq = Wqkv[:, :D_MODEL][:, perm]
    wk = Wqkv[:, D_MODEL:2 * D_MODEL][:, perm]
    wv = Wqkv[:, 2 * D_MODEL:]
    wiw_pad = jnp.pad(Wiw, ((0, 0), (0, 64 - IND_HEADS)))
    wcat = jnp.concatenate([wq, wk, wv, Wiq, Wik, wiw_pad], axis=1)

    theta = 1.0 / (ROPE_B ** (jnp.arange(HALF, dtype=jnp.float32) * 2.0 / D_K))
    ang = jnp.arange(s, dtype=jnp.float32)[:, None] * theta[None, :]
    cos_t = jnp.tile(jnp.cos(ang), (1, N_HEADS))
    sin_t = jnp.tile(jnp.sin(ang), (1, N_HEADS))

    q, kt, v, iq, ikt, iw = _run_proj(x2, wcat, cos_t, sin_t)
    scores, ent_sum = _run_scores(iq, ikt, iw)
    g = _run_gate(x2, Wg1, bg1, Wg2, bg2)

    ent_mean = ent_sum[0, 0] / jnp.float32(s)
    entropy_norm = ent_mean / jnp.log(float(s))
    entropy_factor = jnp.clip(GMIN + entropy_norm, GMIN, GMAX)
    gate_factor = GMIN + (GMAX - GMIN) * g[0, 0]
    scaled_gate = jnp.float32(TOPK_BASE) * gate_factor
    adaptive_k = _exact_floor_prod_(scaled_gate, entropy_factor).astype(jnp.int32)
    adaptive_k = jnp.clip(adaptive_k, 1, s)

    mask = _run_select(scores, adaptive_k.reshape(1, 1))
    y = _run_attn(q, kt, v, mask, Wo)
    return y.reshape(b, s, d)
